# baseline (speedup 1.0000x reference)
import functools

import numpy as np
import jax
import jax.numpy as jnp
from jax import lax
from jax.experimental import pallas as pl
from jax.experimental.pallas import tpu as pltpu
from jax.experimental.pallas import tpu_sc as plsc

TABLE_SIZE = 100000
EMB = 64
NUM_HEADS = 4
HIDDEN = 1024
CONCAT = 512
B, S = 4, 2048
P = B * S

_rng = np.random.RandomState(42)
_M2 = [int(m) % TABLE_SIZE for m in _rng.randint(2, 2 ** 31, size=NUM_HEADS)]
_M3 = [int(m) % TABLE_SIZE for m in _rng.randint(2, 2 ** 31, size=NUM_HEADS)]
_MULTS = _M2 + _M3

_NW = 32
_CHUNK = P // _NW
_NBUF = 3
_WIDE = 2 * EMB


def _mulmod(h, m_mod):
    h_hi = h >> 8
    h_lo = h & 255
    return ((h_hi * m_mod) % TABLE_SIZE * 256 + h_lo * m_mod) % TABLE_SIZE


def _sc_body(tok_hbm, tab_hbm, raw_hbm, par_hbm, tok_v, idx_v, par_v, rows_v,
             sem0, sem1, sem2, wsem):
    wid = lax.axis_index("c") * 16 + lax.axis_index("s")
    base = wid * _CHUNK
    row_start = (base % S) == 0

    pltpu.sync_copy(tok_hbm.at[pl.ds(base, _CHUNK)], tok_v.at[pl.ds(16, _CHUNK)])

    @pl.when(row_start)
    def _():
        tok_v[pl.ds(0, 16)] = jnp.zeros((16,), jnp.int32)

    @pl.when(jnp.logical_not(row_start))
    def _():
        pltpu.sync_copy(tok_hbm.at[pl.ds(base - 16, 16)], tok_v.at[pl.ds(0, 16)])

    def hash_step(i, carry):
        off = i * jnp.int32(16)
        t0 = tok_v[pl.ds(off + 16, 16)] % TABLE_SIZE
        t1 = tok_v[pl.ds(off + 15, 16)] % TABLE_SIZE
        t2 = tok_v[pl.ds(off + 14, 16)] % TABLE_SIZE
        for p in range(8):
            m = _MULTS[p]
            if p < 4:
                h = (_mulmod(t1, m) + t0) % TABLE_SIZE
            else:
                h2 = (_mulmod(t2, m) + t1) % TABLE_SIZE
                h = (_mulmod(h2, m) + t0) % TABLE_SIZE
            g = h + p * TABLE_SIZE
            idx_v[p, pl.ds(off, 16)] = g >> 1
            par_v[p, pl.ds(off, 16)] = (g & 1).astype(jnp.float32)
        return carry

    lax.fori_loop(jnp.int32(0), jnp.int32(_CHUNK // 16), hash_step,
                  jnp.int32(0))

    pltpu.sync_copy(par_v, par_hbm.at[:, pl.ds(base, _CHUNK)])

    sems = (sem0, sem1, sem2)
    gd = {}
    wd = {}

    def fire(p):
        slot = p % _NBUF
        d0 = pltpu.async_copy(
            tab_hbm.at[idx_v.at[jnp.int32(p), pl.ds(0, 128)]],
            rows_v.at[jnp.int32(slot), pl.ds(0, 128)], sems[slot])
        d1 = pltpu.async_copy(
            tab_hbm.at[idx_v.at[jnp.int32(p), pl.ds(128, 128)]],
            rows_v.at[jnp.int32(slot), pl.ds(128, 128)], sems[slot])
        gd[p] = (d0, d1)

    def writeout(p):
        slot = p % _NBUF
        for d in gd[p]:
            d.wait()
        wd[p] = pltpu.async_copy(
            rows_v.at[jnp.int32(slot)],
            raw_hbm.at[jnp.int32(p), pl.ds(base, _CHUNK)], wsem)

    for p in range(8):
        if p >= _NBUF:
            wd[p - _NBUF].wait()
        fire(p)
        if p >= _NBUF - 1:
            writeout(p - _NBUF + 1)
    for p in range(8 - _NBUF + 1, 8):
        writeout(p)
    for p in range(8 - _NBUF, 8):
        wd[p].wait()


def _sc_gather(tok_i32, tab_wide):
    mesh = plsc.VectorSubcoreMesh(
        core_axis_name="c", subcore_axis_name="s", num_cores=2, num_subcores=16)
    return pl.kernel(
        _sc_body,
        out_type=(
            jax.ShapeDtypeStruct((8, P, _WIDE), jnp.float32),
            jax.ShapeDtypeStruct((8, P), jnp.float32),
        ),
        mesh=mesh,
        compiler_params=pltpu.CompilerParams(use_tc_tiling_on_sc=False),
        scratch_types=[
            pltpu.VMEM((16 + _CHUNK,), jnp.int32),
            pltpu.VMEM((8, _CHUNK), jnp.int32),
            pltpu.VMEM((8, _CHUNK), jnp.float32),
            pltpu.VMEM((_NBUF, _CHUNK, _WIDE), jnp.float32),
            pltpu.SemaphoreType.DMA,
            pltpu.SemaphoreType.DMA,
            pltpu.SemaphoreType.DMA,
            pltpu.SemaphoreType.DMA,
        ],
    )(tok_i32, tab_wide)


_BS = 512

_dot = functools.partial(
    jnp.dot, preferred_element_type=jnp.float32,
    precision=lax.Precision.HIGHEST)


def _tc_q_body(hid_ref, gqw_ref, gqb_ref, qng_ref, qnb_ref, q_ref):
    hid = hid_ref[...]
    hmu = jnp.mean(hid, axis=-1, keepdims=True)
    hvar = jnp.mean(jnp.square(hid - hmu), axis=-1, keepdims=True)
    hn = (hid - hmu) * lax.rsqrt(hvar + 1e-5) * qng_ref[...] + qnb_ref[...]
    q_ref[...] = _dot(hn, gqw_ref[...]) + gqb_ref[...]


def _tc_q(hid2, gq_w, gq_b, qn_g, qn_b):
    f32 = jnp.float32
    z = np.int32(0)
    full = lambda shape: pl.BlockSpec(shape, lambda i: (z, z))
    return pl.pallas_call(
        _tc_q_body,
        grid=(P // _BS,),
        in_specs=[
            pl.BlockSpec((_BS, HIDDEN), lambda i: (i, z)),
            full((HIDDEN, EMB)), full((1, EMB)),
            full((1, HIDDEN)), full((1, HIDDEN)),
        ],
        out_specs=pl.BlockSpec((_BS, EMB), lambda i: (i, z)),
        out_shape=jax.ShapeDtypeStruct((P, EMB), f32),
    )(hid2, gq_w.T.astype(f32), gq_b.reshape(1, EMB).astype(f32),
      qn_g.reshape(1, HIDDEN).astype(f32), qn_b.reshape(1, HIDDEN).astype(f32))


def _tc_gate_body(raw_ref, par_ref, q_ref, gkw_ref, gvw_ref, opw_ref,
                  gkb_ref, gvb_ref, opb_ref,
                  rng_ref, rnb_ref, ong_ref, onb_ref,
                  out_ref, gate_ref):
    pid = pl.program_id(0)

    rows = lax.broadcasted_iota(jnp.int32, (_BS, 1), 0)
    pos = (pid * jnp.int32(_BS) + rows) % jnp.int32(S)
    cols = lax.broadcasted_iota(jnp.int32, (1, CONCAT), 1)
    need = jnp.where(cols < jnp.int32(NUM_HEADS * EMB),
                     jnp.int32(1), jnp.int32(2))
    raw8w = raw_ref[...]
    par = par_ref[...]
    parts = []
    for p in range(8):
        x0 = raw8w[p, :, :EMB]
        x1 = raw8w[p, :, EMB:]
        pf = lax.broadcast_in_dim(par[p], (_BS, EMB), (0,))
        parts.append(x0 + (x1 - x0) * pf)
    raw = jnp.concatenate(parts, axis=-1)
    raw = jnp.where(pos >= need, raw, jnp.float32(0.0))

    mu = jnp.mean(raw, axis=-1, keepdims=True)
    var = jnp.mean(jnp.square(raw - mu), axis=-1, keepdims=True)
    rn = (raw - mu) * lax.rsqrt(var + 1e-5) * rng_ref[...] + rnb_ref[...]

    q = q_ref[...]
    k = _dot(rn, gkw_ref[...]) + gkb_ref[...]
    v = _dot(rn, gvw_ref[...]) + gvb_ref[...]

    gate = jax.nn.sigmoid(jnp.sum(q * k, axis=-1, keepdims=True) * (1.0 / 8.0))
    o = _dot(gate * v, opw_ref[...]) + opb_ref[...]

    omu = jnp.mean(o, axis=-1, keepdims=True)
    ovar = jnp.mean(jnp.square(o - omu), axis=-1, keepdims=True)
    out_ref[...] = (o - omu) * lax.rsqrt(ovar + 1e-5) * ong_ref[...] + onb_ref[...]
    gate_ref[...] = gate


def _tc_gated(raw, par, q, gk_w, gk_b, gv_w, gv_b,
              rn_g, rn_b, op_w, op_b, on_g, on_b):
    f32 = jnp.float32
    z = np.int32(0)
    full = lambda shape: pl.BlockSpec(shape, lambda i: (z, z))
    return pl.pallas_call(
        _tc_gate_body,
        grid=(P // _BS,),
        in_specs=[
            pl.BlockSpec((8, _BS, _WIDE), lambda i: (z, i, z)),
            pl.BlockSpec((8, _BS), lambda i: (z, i)),
            pl.BlockSpec((_BS, EMB), lambda i: (i, z)),
            full((CONCAT, EMB)), full((CONCAT, EMB)), full((EMB, EMB)),
            full((1, EMB)), full((1, EMB)), full((1, EMB)),
            full((1, CONCAT)), full((1, CONCAT)),
            full((1, EMB)), full((1, EMB)),
        ],
        out_specs=(
            pl.BlockSpec((_BS, EMB), lambda i: (i, z)),
            pl.BlockSpec((_BS, 1), lambda i: (i, z)),
        ),
        out_shape=(
            jax.ShapeDtypeStruct((P, EMB), f32),
            jax.ShapeDtypeStruct((P, 1), f32),
        ),
    )(raw, par, q,
      gk_w.T.astype(f32), gv_w.T.astype(f32), op_w.T.astype(f32),
      gk_b.reshape(1, EMB).astype(f32), gv_b.reshape(1, EMB).astype(f32),
      op_b.reshape(1, EMB).astype(f32),
      rn_g.reshape(1, CONCAT).astype(f32), rn_b.reshape(1, CONCAT).astype(f32),
      on_g.reshape(1, EMB).astype(f32), on_b.reshape(1, EMB).astype(f32))


def kernel(token_ids, hidden_states, tables, gq_w, gq_b, gk_w, gk_b,
           gv_w, gv_b, rn_g, rn_b, qn_g, qn_b, op_w, op_b, on_g, on_b):
    tok_i32 = token_ids.astype(jnp.int32).reshape(P)
    tab_wide = jnp.reshape(tables.astype(jnp.float32),
                           (len(_MULTS) * TABLE_SIZE // 2, 2 * EMB))
    raw, par = _sc_gather(tok_i32, tab_wide)
    hid2 = hidden_states.astype(jnp.float32).reshape(P, HIDDEN)
    q = _tc_q(hid2, gq_w, gq_b, qn_g, qn_b)
    mem, gate = _tc_gated(raw, par, q, gk_w, gk_b, gv_w, gv_b,
                          rn_g, rn_b, op_w, op_b, on_g, on_b)
    return (mem.reshape(B, S, EMB).astype(jnp.float64),
            gate.reshape(B, S, 1).astype(jnp.float64))


# speedup vs baseline: 1.0053x; 1.0053x over previous
import functools

import numpy as np
import jax
import jax.numpy as jnp
from jax import lax
from jax.experimental import pallas as pl
from jax.experimental.pallas import tpu as pltpu
from jax.experimental.pallas import tpu_sc as plsc

TABLE_SIZE = 100000
EMB = 64
NUM_HEADS = 4
HIDDEN = 1024
CONCAT = 512
B, S = 4, 2048
P = B * S

_rng = np.random.RandomState(42)
_M2 = [int(m) % TABLE_SIZE for m in _rng.randint(2, 2 ** 31, size=NUM_HEADS)]
_M3 = [int(m) % TABLE_SIZE for m in _rng.randint(2, 2 ** 31, size=NUM_HEADS)]
_MULTS = _M2 + _M3

_NW = 32
_CHUNK = P // _NW
_NBUF = 3
_WIDE = 2 * EMB


def _mulmod(h, m_mod):
    h_hi = h >> 8
    h_lo = h & 255
    return ((h_hi * m_mod) % TABLE_SIZE * 256 + h_lo * m_mod) % TABLE_SIZE


def _sc_body(tok_hbm, tab_hbm, raw_hbm, par_hbm, tok_v, idx_v, par_v, rows_v,
             sem0, sem1, sem2, wsem):
    wid = lax.axis_index("c") * 16 + lax.axis_index("s")
    base = wid * _CHUNK
    row_start = (base % S) == 0

    pltpu.sync_copy(tok_hbm.at[pl.ds(base, _CHUNK)], tok_v.at[pl.ds(16, _CHUNK)])

    @pl.when(row_start)
    def _():
        tok_v[pl.ds(0, 16)] = jnp.zeros((16,), jnp.int32)

    @pl.when(jnp.logical_not(row_start))
    def _():
        pltpu.sync_copy(tok_hbm.at[pl.ds(base - 16, 16)], tok_v.at[pl.ds(0, 16)])

    def hash_step(i, carry):
        off = i * jnp.int32(16)
        t0 = tok_v[pl.ds(off + 16, 16)] % TABLE_SIZE
        t1 = tok_v[pl.ds(off + 15, 16)] % TABLE_SIZE
        t2 = tok_v[pl.ds(off + 14, 16)] % TABLE_SIZE
        for p in range(8):
            m = _MULTS[p]
            if p < 4:
                h = (_mulmod(t1, m) + t0) % TABLE_SIZE
            else:
                h2 = (_mulmod(t2, m) + t1) % TABLE_SIZE
                h = (_mulmod(h2, m) + t0) % TABLE_SIZE
            g = h + p * TABLE_SIZE
            idx_v[pl.ds(p * _CHUNK + off, 16)] = g >> 1
            par_v[pl.ds(p * _CHUNK + off, 16)] = (g & 1).astype(jnp.float32)
        return carry

    lax.fori_loop(jnp.int32(0), jnp.int32(_CHUNK // 16), hash_step,
                  jnp.int32(0))

    for p in range(8):
        pltpu.sync_copy(par_v.at[pl.ds(p * _CHUNK, _CHUNK)],
                        par_hbm.at[jnp.int32(p), pl.ds(base, _CHUNK)])

    sems = (sem0, sem1, sem2)
    gd = {}
    wd = {}

    def fire(p):
        slot = p % _NBUF
        d0 = pltpu.async_copy(
            tab_hbm.at[idx_v.at[pl.ds(p * _CHUNK, 128)]],
            rows_v.at[jnp.int32(slot), pl.ds(0, 128)], sems[slot])
        d1 = pltpu.async_copy(
            tab_hbm.at[idx_v.at[pl.ds(p * _CHUNK + 128, 128)]],
            rows_v.at[jnp.int32(slot), pl.ds(128, 128)], sems[slot])
        gd[p] = (d0, d1)

    def writeout(p):
        slot = p % _NBUF
        for d in gd[p]:
            d.wait()
        wd[p] = pltpu.async_copy(
            rows_v.at[jnp.int32(slot)],
            raw_hbm.at[jnp.int32(p), pl.ds(base, _CHUNK)], wsem)

    for p in range(8):
        if p >= _NBUF:
            wd[p - _NBUF].wait()
        fire(p)
        if p >= _NBUF - 1:
            writeout(p - _NBUF + 1)
    for p in range(8 - _NBUF + 1, 8):
        writeout(p)
    for p in range(8 - _NBUF, 8):
        wd[p].wait()


def _sc_gather(tok_i32, tab_wide):
    mesh = plsc.VectorSubcoreMesh(
        core_axis_name="c", subcore_axis_name="s", num_cores=2, num_subcores=16)
    return pl.kernel(
        _sc_body,
        out_type=(
            jax.ShapeDtypeStruct((8, P, _WIDE), jnp.float32),
            jax.ShapeDtypeStruct((8, P), jnp.float32),
        ),
        mesh=mesh,
        compiler_params=pltpu.CompilerParams(use_tc_tiling_on_sc=True),
        scratch_types=[
            pltpu.VMEM((16 + _CHUNK,), jnp.int32),
            pltpu.VMEM((8 * _CHUNK,), jnp.int32),
            pltpu.VMEM((8 * _CHUNK,), jnp.float32),
            pltpu.VMEM((_NBUF, _CHUNK, _WIDE), jnp.float32),
            pltpu.SemaphoreType.DMA,
            pltpu.SemaphoreType.DMA,
            pltpu.SemaphoreType.DMA,
            pltpu.SemaphoreType.DMA,
        ],
    )(tok_i32, tab_wide)


_BS = 512

_dot = functools.partial(
    jnp.dot, preferred_element_type=jnp.float32,
    precision=lax.Precision.HIGHEST)


def _tc_q_body(hid_ref, gqw_ref, gqb_ref, qng_ref, qnb_ref, q_ref):
    hid = hid_ref[...]
    hmu = jnp.mean(hid, axis=-1, keepdims=True)
    hvar = jnp.mean(jnp.square(hid - hmu), axis=-1, keepdims=True)
    hn = (hid - hmu) * lax.rsqrt(hvar + 1e-5) * qng_ref[...] + qnb_ref[...]
    q_ref[...] = _dot(hn, gqw_ref[...]) + gqb_ref[...]


def _tc_q(hid2, gq_w, gq_b, qn_g, qn_b):
    f32 = jnp.float32
    z = np.int32(0)
    full = lambda shape: pl.BlockSpec(shape, lambda i: (z, z))
    return pl.pallas_call(
        _tc_q_body,
        grid=(P // _BS,),
        in_specs=[
            pl.BlockSpec((_BS, HIDDEN), lambda i: (i, z)),
            full((HIDDEN, EMB)), full((1, EMB)),
            full((1, HIDDEN)), full((1, HIDDEN)),
        ],
        out_specs=pl.BlockSpec((_BS, EMB), lambda i: (i, z)),
        out_shape=jax.ShapeDtypeStruct((P, EMB), f32),
    )(hid2, gq_w.T.astype(f32), gq_b.reshape(1, EMB).astype(f32),
      qn_g.reshape(1, HIDDEN).astype(f32), qn_b.reshape(1, HIDDEN).astype(f32))


def _tc_gate_body(raw_ref, par_ref, q_ref, gkw_ref, gvw_ref, opw_ref,
                  gkb_ref, gvb_ref, opb_ref,
                  rng_ref, rnb_ref, ong_ref, onb_ref,
                  out_ref, gate_ref):
    pid = pl.program_id(0)

    rows = lax.broadcasted_iota(jnp.int32, (_BS, 1), 0)
    pos = (pid * jnp.int32(_BS) + rows) % jnp.int32(S)
    cols = lax.broadcasted_iota(jnp.int32, (1, CONCAT), 1)
    need = jnp.where(cols < jnp.int32(NUM_HEADS * EMB),
                     jnp.int32(1), jnp.int32(2))
    raw8w = raw_ref[...]
    par = par_ref[...]
    parts = []
    for p in range(8):
        x0 = raw8w[p, :, :EMB]
        x1 = raw8w[p, :, EMB:]
        pf = lax.broadcast_in_dim(par[p], (_BS, EMB), (0,))
        parts.append(x0 + (x1 - x0) * pf)
    raw = jnp.concatenate(parts, axis=-1)
    raw = jnp.where(pos >= need, raw, jnp.float32(0.0))

    mu = jnp.mean(raw, axis=-1, keepdims=True)
    var = jnp.mean(jnp.square(raw - mu), axis=-1, keepdims=True)
    rn = (raw - mu) * lax.rsqrt(var + 1e-5) * rng_ref[...] + rnb_ref[...]

    q = q_ref[...]
    k = _dot(rn, gkw_ref[...]) + gkb_ref[...]
    v = _dot(rn, gvw_ref[...]) + gvb_ref[...]

    gate = jax.nn.sigmoid(jnp.sum(q * k, axis=-1, keepdims=True) * (1.0 / 8.0))
    o = _dot(gate * v, opw_ref[...]) + opb_ref[...]

    omu = jnp.mean(o, axis=-1, keepdims=True)
    ovar = jnp.mean(jnp.square(o - omu), axis=-1, keepdims=True)
    out_ref[...] = (o - omu) * lax.rsqrt(ovar + 1e-5) * ong_ref[...] + onb_ref[...]
    gate_ref[...] = gate


def _tc_gated(raw, par, q, gk_w, gk_b, gv_w, gv_b,
              rn_g, rn_b, op_w, op_b, on_g, on_b):
    f32 = jnp.float32
    z = np.int32(0)
    full = lambda shape: pl.BlockSpec(shape, lambda i: (z, z))
    return pl.pallas_call(
        _tc_gate_body,
        grid=(P // _BS,),
        in_specs=[
            pl.BlockSpec((8, _BS, _WIDE), lambda i: (z, i, z)),
            pl.BlockSpec((8, _BS), lambda i: (z, i)),
            pl.BlockSpec((_BS, EMB), lambda i: (i, z)),
            full((CONCAT, EMB)), full((CONCAT, EMB)), full((EMB, EMB)),
            full((1, EMB)), full((1, EMB)), full((1, EMB)),
            full((1, CONCAT)), full((1, CONCAT)),
            full((1, EMB)), full((1, EMB)),
        ],
        out_specs=(
            pl.BlockSpec((_BS, EMB), lambda i: (i, z)),
            pl.BlockSpec((_BS, 1), lambda i: (i, z)),
        ),
        out_shape=(
            jax.ShapeDtypeStruct((P, EMB), f32),
            jax.ShapeDtypeStruct((P, 1), f32),
        ),
    )(raw, par, q,
      gk_w.T.astype(f32), gv_w.T.astype(f32), op_w.T.astype(f32),
      gk_b.reshape(1, EMB).astype(f32), gv_b.reshape(1, EMB).astype(f32),
      op_b.reshape(1, EMB).astype(f32),
      rn_g.reshape(1, CONCAT).astype(f32), rn_b.reshape(1, CONCAT).astype(f32),
      on_g.reshape(1, EMB).astype(f32), on_b.reshape(1, EMB).astype(f32))


def kernel(token_ids, hidden_states, tables, gq_w, gq_b, gk_w, gk_b,
           gv_w, gv_b, rn_g, rn_b, qn_g, qn_b, op_w, op_b, on_g, on_b):
    tok_i32 = token_ids.astype(jnp.int32).reshape(P)
    tab_wide = jnp.reshape(tables.astype(jnp.float32),
                           (len(_MULTS) * TABLE_SIZE // 2, 2 * EMB))
    raw, par = _sc_gather(tok_i32, tab_wide)
    hid2 = hidden_states.astype(jnp.float32).reshape(P, HIDDEN)
    q = _tc_q(hid2, gq_w, gq_b, qn_g, qn_b)
    mem, gate = _tc_gated(raw, par, q, gk_w, gk_b, gv_w, gv_b,
                          rn_g, rn_b, op_w, op_b, on_g, on_b)
    return (mem.reshape(B, S, EMB).astype(jnp.float64),
            gate.reshape(B, S, 1).astype(jnp.float64))


# speedup vs baseline: 1.0464x; 1.0409x over previous
import functools

import numpy as np
import jax
import jax.numpy as jnp
from jax import lax
from jax.experimental import pallas as pl
from jax.experimental.pallas import tpu as pltpu
from jax.experimental.pallas import tpu_sc as plsc

TABLE_SIZE = 100000
EMB = 64
NUM_HEADS = 4
HIDDEN = 1024
CONCAT = 512
B, S = 4, 2048
P = B * S

_rng = np.random.RandomState(42)
_M2 = [int(m) % TABLE_SIZE for m in _rng.randint(2, 2 ** 31, size=NUM_HEADS)]
_M3 = [int(m) % TABLE_SIZE for m in _rng.randint(2, 2 ** 31, size=NUM_HEADS)]
_MULTS = _M2 + _M3

_NW = 32
_CHUNK = P // _NW
_NBUF = 4


def _mulmod(h, m_mod):
    h_hi = h >> 8
    h_lo = h & 255
    return ((h_hi * m_mod) % TABLE_SIZE * 256 + h_lo * m_mod) % TABLE_SIZE


def _sc_body(tok_hbm, tab_hbm, raw_hbm, tok_v, idx_v, rows_v,
             sem0, sem1, sem2, sem3, wsem):
    wid = lax.axis_index("c") * 16 + lax.axis_index("s")
    base = wid * _CHUNK
    row_start = (base % S) == 0

    pltpu.sync_copy(tok_hbm.at[pl.ds(base, _CHUNK)], tok_v.at[pl.ds(16, _CHUNK)])

    @pl.when(row_start)
    def _():
        tok_v[pl.ds(0, 16)] = jnp.zeros((16,), jnp.int32)

    @pl.when(jnp.logical_not(row_start))
    def _():
        pltpu.sync_copy(tok_hbm.at[pl.ds(base - 16, 16)], tok_v.at[pl.ds(0, 16)])

    def hash_step(i, carry):
        off = i * jnp.int32(16)
        t0 = tok_v[pl.ds(off + 16, 16)] % TABLE_SIZE
        t1 = tok_v[pl.ds(off + 15, 16)] % TABLE_SIZE
        t2 = tok_v[pl.ds(off + 14, 16)] % TABLE_SIZE
        for p in range(8):
            m = _MULTS[p]
            if p < 4:
                h = (_mulmod(t1, m) + t0) % TABLE_SIZE
            else:
                h2 = (_mulmod(t2, m) + t1) % TABLE_SIZE
                h = (_mulmod(h2, m) + t0) % TABLE_SIZE
            idx_v[p, pl.ds(off, 16)] = h + p * TABLE_SIZE
        return carry

    lax.fori_loop(jnp.int32(0), jnp.int32(_CHUNK // 16), hash_step,
                  jnp.int32(0))

    sems = (sem0, sem1, sem2, sem3)
    gd = {}
    wd = {}

    def fire(p):
        slot = p % _NBUF
        d0 = pltpu.async_copy(
            tab_hbm.at[idx_v.at[jnp.int32(p), pl.ds(0, 128)]],
            rows_v.at[jnp.int32(slot), pl.ds(0, 128)], sems[slot])
        d1 = pltpu.async_copy(
            tab_hbm.at[idx_v.at[jnp.int32(p), pl.ds(128, 128)]],
            rows_v.at[jnp.int32(slot), pl.ds(128, 128)], sems[slot])
        gd[p] = (d0, d1)

    def writeout(p):
        slot = p % _NBUF
        for d in gd[p]:
            d.wait()
        wd[p] = pltpu.async_copy(
            rows_v.at[jnp.int32(slot)],
            raw_hbm.at[jnp.int32(p), pl.ds(base, _CHUNK)], wsem)

    for p in range(8):
        if p >= _NBUF:
            wd[p - _NBUF].wait()
        fire(p)
        if p >= _NBUF - 1:
            writeout(p - _NBUF + 1)
    for p in range(8 - _NBUF + 1, 8):
        writeout(p)
    for p in range(8 - _NBUF, 8):
        wd[p].wait()


def _sc_gather(tok_i32, tab_flat):
    mesh = plsc.VectorSubcoreMesh(
        core_axis_name="c", subcore_axis_name="s", num_cores=2, num_subcores=16)
    return pl.kernel(
        _sc_body,
        out_type=jax.ShapeDtypeStruct((8, P, EMB), jnp.float32),
        mesh=mesh,
        compiler_params=pltpu.CompilerParams(use_tc_tiling_on_sc=False),
        scratch_types=[
            pltpu.VMEM((16 + _CHUNK,), jnp.int32),
            pltpu.VMEM((8, _CHUNK), jnp.int32),
            pltpu.VMEM((_NBUF, _CHUNK, EMB), jnp.float32),
            pltpu.SemaphoreType.DMA,
            pltpu.SemaphoreType.DMA,
            pltpu.SemaphoreType.DMA,
            pltpu.SemaphoreType.DMA,
            pltpu.SemaphoreType.DMA,
        ],
    )(tok_i32, tab_flat)


_BS = 512

_dot = functools.partial(
    jnp.dot, preferred_element_type=jnp.float32,
    precision=lax.Precision.HIGHEST)


def _tc_q_body(hid_ref, gqw_ref, gqb_ref, qng_ref, qnb_ref, q_ref):
    hid = hid_ref[...]
    hmu = jnp.mean(hid, axis=-1, keepdims=True)
    hvar = jnp.mean(jnp.square(hid - hmu), axis=-1, keepdims=True)
    hn = (hid - hmu) * lax.rsqrt(hvar + 1e-5) * qng_ref[...] + qnb_ref[...]
    q_ref[...] = _dot(hn, gqw_ref[...]) + gqb_ref[...]


def _tc_q(hid2, gq_w, gq_b, qn_g, qn_b):
    f32 = jnp.float32
    z = np.int32(0)
    full = lambda shape: pl.BlockSpec(shape, lambda i: (z, z))
    return pl.pallas_call(
        _tc_q_body,
        grid=(P // _BS,),
        in_specs=[
            pl.BlockSpec((_BS, HIDDEN), lambda i: (i, z)),
            full((HIDDEN, EMB)), full((1, EMB)),
            full((1, HIDDEN)), full((1, HIDDEN)),
        ],
        out_specs=pl.BlockSpec((_BS, EMB), lambda i: (i, z)),
        out_shape=jax.ShapeDtypeStruct((P, EMB), f32),
    )(hid2, gq_w.T.astype(f32), gq_b.reshape(1, EMB).astype(f32),
      qn_g.reshape(1, HIDDEN).astype(f32), qn_b.reshape(1, HIDDEN).astype(f32))


def _tc_gate_body(raw_ref, q_ref, gkw_ref, gvw_ref, opw_ref,
                  gkb_ref, gvb_ref, opb_ref,
                  rng_ref, rnb_ref, ong_ref, onb_ref,
                  out_ref, gate_ref):
    pid = pl.program_id(0)

    rows = lax.broadcasted_iota(jnp.int32, (_BS, 1), 0)
    pos = (pid * jnp.int32(_BS) + rows) % jnp.int32(S)
    cols = lax.broadcasted_iota(jnp.int32, (1, CONCAT), 1)
    need = jnp.where(cols < jnp.int32(NUM_HEADS * EMB),
                     jnp.int32(1), jnp.int32(2))
    raw8 = raw_ref[...]
    raw = jnp.concatenate([raw8[p] for p in range(8)], axis=-1)
    raw = jnp.where(pos >= need, raw, jnp.float32(0.0))

    mu = jnp.mean(raw, axis=-1, keepdims=True)
    var = jnp.mean(jnp.square(raw - mu), axis=-1, keepdims=True)
    rn = (raw - mu) * lax.rsqrt(var + 1e-5) * rng_ref[...] + rnb_ref[...]

    q = q_ref[...]
    k = _dot(rn, gkw_ref[...]) + gkb_ref[...]
    v = _dot(rn, gvw_ref[...]) + gvb_ref[...]

    gate = jax.nn.sigmoid(jnp.sum(q * k, axis=-1, keepdims=True) * (1.0 / 8.0))
    o = _dot(gate * v, opw_ref[...]) + opb_ref[...]

    omu = jnp.mean(o, axis=-1, keepdims=True)
    ovar = jnp.mean(jnp.square(o - omu), axis=-1, keepdims=True)
    out_ref[...] = (o - omu) * lax.rsqrt(ovar + 1e-5) * ong_ref[...] + onb_ref[...]
    gate_ref[...] = gate


def _tc_gated(raw, q, gk_w, gk_b, gv_w, gv_b,
              rn_g, rn_b, op_w, op_b, on_g, on_b):
    f32 = jnp.float32
    z = np.int32(0)
    full = lambda shape: pl.BlockSpec(shape, lambda i: (z, z))
    return pl.pallas_call(
        _tc_gate_body,
        grid=(P // _BS,),
        in_specs=[
            pl.BlockSpec((8, _BS, EMB), lambda i: (z, i, z)),
            pl.BlockSpec((_BS, EMB), lambda i: (i, z)),
            full((CONCAT, EMB)), full((CONCAT, EMB)), full((EMB, EMB)),
            full((1, EMB)), full((1, EMB)), full((1, EMB)),
            full((1, CONCAT)), full((1, CONCAT)),
            full((1, EMB)), full((1, EMB)),
        ],
        out_specs=(
            pl.BlockSpec((_BS, EMB), lambda i: (i, z)),
            pl.BlockSpec((_BS, 1), lambda i: (i, z)),
        ),
        out_shape=(
            jax.ShapeDtypeStruct((P, EMB), f32),
            jax.ShapeDtypeStruct((P, 1), f32),
        ),
    )(raw, q,
      gk_w.T.astype(f32), gv_w.T.astype(f32), op_w.T.astype(f32),
      gk_b.reshape(1, EMB).astype(f32), gv_b.reshape(1, EMB).astype(f32),
      op_b.reshape(1, EMB).astype(f32),
      rn_g.reshape(1, CONCAT).astype(f32), rn_b.reshape(1, CONCAT).astype(f32),
      on_g.reshape(1, EMB).astype(f32), on_b.reshape(1, EMB).astype(f32))


def kernel(token_ids, hidden_states, tables, gq_w, gq_b, gk_w, gk_b,
           gv_w, gv_b, rn_g, rn_b, qn_g, qn_b, op_w, op_b, on_g, on_b):
    tok_i32 = token_ids.astype(jnp.int32).reshape(P)
    tab_flat = tables.astype(jnp.float32).reshape(
        len(_MULTS) * TABLE_SIZE, EMB)
    raw = _sc_gather(tok_i32, tab_flat)
    hid2 = hidden_states.astype(jnp.float32).reshape(P, HIDDEN)
    q = _tc_q(hid2, gq_w, gq_b, qn_g, qn_b)
    mem, gate = _tc_gated(raw, q, gk_w, gk_b, gv_w, gv_b,
                          rn_g, rn_b, op_w, op_b, on_g, on_b)
    return (mem.reshape(B, S, EMB).astype(jnp.float64),
            gate.reshape(B, S, 1).astype(jnp.float64))


# speedup vs baseline: 1.1029x; 1.0540x over previous
import functools

import numpy as np
import jax
import jax.numpy as jnp
from jax import lax
from jax.experimental import pallas as pl
from jax.experimental.pallas import tpu as pltpu
from jax.experimental.pallas import tpu_sc as plsc

TABLE_SIZE = 100000
EMB = 64
NUM_HEADS = 4
HIDDEN = 1024
CONCAT = 512
B, S = 4, 2048
P = B * S

_rng = np.random.RandomState(42)
_M2 = [int(m) % TABLE_SIZE for m in _rng.randint(2, 2 ** 31, size=NUM_HEADS)]
_M3 = [int(m) % TABLE_SIZE for m in _rng.randint(2, 2 ** 31, size=NUM_HEADS)]
_MULTS = _M2 + _M3

_NW = 32
_CHUNK = P // _NW
_NBUF = 4


def _mulmod(h, m_mod):
    h_hi = h >> 8
    h_lo = h & 255
    return ((h_hi * m_mod) % TABLE_SIZE * 256 + h_lo * m_mod) % TABLE_SIZE


def _sc_body(tok_hbm, tab_hbm, raw_hbm, tok_v, idx_v, rows_v,
             sem0, sem1, sem2, sem3, wsem):
    wid = lax.axis_index("c") * 16 + lax.axis_index("s")
    base = wid * _CHUNK
    row_start = (base % S) == 0

    pltpu.sync_copy(tok_hbm.at[pl.ds(base, _CHUNK)], tok_v.at[pl.ds(16, _CHUNK)])

    @pl.when(row_start)
    def _():
        tok_v[pl.ds(0, 16)] = jnp.zeros((16,), jnp.int32)

    @pl.when(jnp.logical_not(row_start))
    def _():
        pltpu.sync_copy(tok_hbm.at[pl.ds(base - 16, 16)], tok_v.at[pl.ds(0, 16)])

    def hash_step(i, carry):
        off = i * jnp.int32(16)
        t0 = tok_v[pl.ds(off + 16, 16)] % TABLE_SIZE
        t1 = tok_v[pl.ds(off + 15, 16)] % TABLE_SIZE
        t2 = tok_v[pl.ds(off + 14, 16)] % TABLE_SIZE
        for p in range(8):
            m = _MULTS[p]
            if p < 4:
                h = (_mulmod(t1, m) + t0) % TABLE_SIZE
            else:
                h2 = (_mulmod(t2, m) + t1) % TABLE_SIZE
                h = (_mulmod(h2, m) + t0) % TABLE_SIZE
            idx_v[p, pl.ds(off, 16)] = h + p * TABLE_SIZE
        return carry

    lax.fori_loop(jnp.int32(0), jnp.int32(_CHUNK // 16), hash_step,
                  jnp.int32(0))

    sems = (sem0, sem1, sem2, sem3)
    gd = {}
    wd = {}

    def fire(p):
        slot = p % _NBUF
        d0 = pltpu.async_copy(
            tab_hbm.at[idx_v.at[jnp.int32(p), pl.ds(0, 128)]],
            rows_v.at[jnp.int32(slot), pl.ds(0, 128)], sems[slot])
        d1 = pltpu.async_copy(
            tab_hbm.at[idx_v.at[jnp.int32(p), pl.ds(128, 128)]],
            rows_v.at[jnp.int32(slot), pl.ds(128, 128)], sems[slot])
        gd[p] = (d0, d1)

    def writeout(p):
        slot = p % _NBUF
        for d in gd[p]:
            d.wait()
        wd[p] = pltpu.async_copy(
            rows_v.at[jnp.int32(slot)],
            raw_hbm.at[jnp.int32(p), pl.ds(base, _CHUNK)], wsem)

    for p in range(8):
        if p >= _NBUF:
            wd[p - _NBUF].wait()
        fire(p)
        if p >= _NBUF - 1:
            writeout(p - _NBUF + 1)
    for p in range(8 - _NBUF + 1, 8):
        writeout(p)
    for p in range(8 - _NBUF, 8):
        wd[p].wait()


def _sc_gather(tok_i32, tab_flat):
    mesh = plsc.VectorSubcoreMesh(
        core_axis_name="c", subcore_axis_name="s", num_cores=2, num_subcores=16)
    return pl.kernel(
        _sc_body,
        out_type=jax.ShapeDtypeStruct((8, P, EMB), jnp.float32),
        mesh=mesh,
        compiler_params=pltpu.CompilerParams(use_tc_tiling_on_sc=False),
        scratch_types=[
            pltpu.VMEM((16 + _CHUNK,), jnp.int32),
            pltpu.VMEM((8, _CHUNK), jnp.int32),
            pltpu.VMEM((_NBUF, _CHUNK, EMB), jnp.float32),
            pltpu.SemaphoreType.DMA,
            pltpu.SemaphoreType.DMA,
            pltpu.SemaphoreType.DMA,
            pltpu.SemaphoreType.DMA,
            pltpu.SemaphoreType.DMA,
        ],
    )(tok_i32, tab_flat)


_BS = 1024

_dot = functools.partial(
    jnp.dot, preferred_element_type=jnp.float32)


def _tc_q_body(hid_ref, gqw_ref, gqb_ref, qng_ref, qnb_ref, q_ref):
    hid = hid_ref[...]
    hmu = jnp.mean(hid, axis=-1, keepdims=True)
    hvar = jnp.mean(jnp.square(hid - hmu), axis=-1, keepdims=True)
    hn = (hid - hmu) * lax.rsqrt(hvar + 1e-5) * qng_ref[...] + qnb_ref[...]
    q_ref[...] = _dot(hn, gqw_ref[...]) + gqb_ref[...]


def _tc_q(hid2, gq_w, gq_b, qn_g, qn_b):
    f32 = jnp.float32
    z = np.int32(0)
    full = lambda shape: pl.BlockSpec(shape, lambda i: (z, z))
    return pl.pallas_call(
        _tc_q_body,
        grid=(P // _BS,),
        in_specs=[
            pl.BlockSpec((_BS, HIDDEN), lambda i: (i, z)),
            full((HIDDEN, EMB)), full((1, EMB)),
            full((1, HIDDEN)), full((1, HIDDEN)),
        ],
        out_specs=pl.BlockSpec((_BS, EMB), lambda i: (i, z)),
        out_shape=jax.ShapeDtypeStruct((P, EMB), f32),
    )(hid2, gq_w.T.astype(f32), gq_b.reshape(1, EMB).astype(f32),
      qn_g.reshape(1, HIDDEN).astype(f32), qn_b.reshape(1, HIDDEN).astype(f32))


def _tc_gate_body(raw_ref, q_ref, gkw_ref, gvw_ref, opw_ref,
                  gkb_ref, gvb_ref, opb_ref,
                  rng_ref, rnb_ref, ong_ref, onb_ref,
                  out_ref, gate_ref):
    pid = pl.program_id(0)

    rows = lax.broadcasted_iota(jnp.int32, (_BS, 1), 0)
    pos = (pid * jnp.int32(_BS) + rows) % jnp.int32(S)
    cols = lax.broadcasted_iota(jnp.int32, (1, CONCAT), 1)
    need = jnp.where(cols < jnp.int32(NUM_HEADS * EMB),
                     jnp.int32(1), jnp.int32(2))
    raw8 = raw_ref[...]
    raw = jnp.concatenate([raw8[p] for p in range(8)], axis=-1)
    raw = jnp.where(pos >= need, raw, jnp.float32(0.0))

    mu = jnp.mean(raw, axis=-1, keepdims=True)
    var = jnp.mean(jnp.square(raw - mu), axis=-1, keepdims=True)
    rn = (raw - mu) * lax.rsqrt(var + 1e-5) * rng_ref[...] + rnb_ref[...]

    q = q_ref[...]
    k = _dot(rn, gkw_ref[...]) + gkb_ref[...]
    v = _dot(rn, gvw_ref[...]) + gvb_ref[...]

    gate = jax.nn.sigmoid(jnp.sum(q * k, axis=-1, keepdims=True) * (1.0 / 8.0))
    o = _dot(gate * v, opw_ref[...]) + opb_ref[...]

    omu = jnp.mean(o, axis=-1, keepdims=True)
    ovar = jnp.mean(jnp.square(o - omu), axis=-1, keepdims=True)
    out_ref[...] = (o - omu) * lax.rsqrt(ovar + 1e-5) * ong_ref[...] + onb_ref[...]
    gate_ref[...] = gate


def _tc_gated(raw, q, gk_w, gk_b, gv_w, gv_b,
              rn_g, rn_b, op_w, op_b, on_g, on_b):
    f32 = jnp.float32
    z = np.int32(0)
    full = lambda shape: pl.BlockSpec(shape, lambda i: (z, z))
    return pl.pallas_call(
        _tc_gate_body,
        grid=(P // _BS,),
        in_specs=[
            pl.BlockSpec((8, _BS, EMB), lambda i: (z, i, z)),
            pl.BlockSpec((_BS, EMB), lambda i: (i, z)),
            full((CONCAT, EMB)), full((CONCAT, EMB)), full((EMB, EMB)),
            full((1, EMB)), full((1, EMB)), full((1, EMB)),
            full((1, CONCAT)), full((1, CONCAT)),
            full((1, EMB)), full((1, EMB)),
        ],
        out_specs=(
            pl.BlockSpec((_BS, EMB), lambda i: (i, z)),
            pl.BlockSpec((_BS, 1), lambda i: (i, z)),
        ),
        out_shape=(
            jax.ShapeDtypeStruct((P, EMB), f32),
            jax.ShapeDtypeStruct((P, 1), f32),
        ),
    )(raw, q,
      gk_w.T.astype(f32), gv_w.T.astype(f32), op_w.T.astype(f32),
      gk_b.reshape(1, EMB).astype(f32), gv_b.reshape(1, EMB).astype(f32),
      op_b.reshape(1, EMB).astype(f32),
      rn_g.reshape(1, CONCAT).astype(f32), rn_b.reshape(1, CONCAT).astype(f32),
      on_g.reshape(1, EMB).astype(f32), on_b.reshape(1, EMB).astype(f32))


def kernel(token_ids, hidden_states, tables, gq_w, gq_b, gk_w, gk_b,
           gv_w, gv_b, rn_g, rn_b, qn_g, qn_b, op_w, op_b, on_g, on_b):
    tok_i32 = token_ids.astype(jnp.int32).reshape(P)
    tab_flat = tables.astype(jnp.float32).reshape(
        len(_MULTS) * TABLE_SIZE, EMB)
    raw = _sc_gather(tok_i32, tab_flat)
    hid2 = hidden_states.astype(jnp.float32).reshape(P, HIDDEN)
    q = _tc_q(hid2, gq_w, gq_b, qn_g, qn_b)
    mem, gate = _tc_gated(raw, q, gk_w, gk_b, gv_w, gv_b,
                          rn_g, rn_b, op_w, op_b, on_g, on_b)
    return (mem.reshape(B, S, EMB).astype(jnp.float64),
            gate.reshape(B, S, 1).astype(jnp.float64))
